# baseline (speedup 1.0000x reference)
import functools
import math

import jax
import jax.numpy as jnp
from jax import lax
from jax.experimental import pallas as pl
from jax.experimental.pallas import tpu as pltpu
from jax.experimental.pallas import tpu_sc as plsc

D_MODEL = 64
SCALE = math.sqrt(D_MODEL)

NUM_CORES = 2
NUM_SUBCORES = 16
NUM_LANES = 16
NW = NUM_CORES * NUM_SUBCORES

CHUNK = 128
NBUF = 4
VECS = D_MODEL // NUM_LANES
UNROLL = 2


def _emb_kernel(n_chunks, idx_hbm, table_hbm, out_hbm, idx_v, rows, tiles, gsems, ssems):
    wid = lax.axis_index("s") * NUM_CORES + lax.axis_index("c")
    chunk_base = wid * n_chunks

    pltpu.sync_copy(idx_hbm.at[pl.ds(chunk_base, n_chunks)], idx_v)

    @plsc.parallel_loop(0, n_chunks, unroll=4)
    def _(g):
        for jd in range(CHUNK // NUM_LANES):
            sl = pl.ds(jd * NUM_LANES, NUM_LANES)
            idx_v[g, sl] = idx_v[g, sl] * 2

    def gather(c, b):
        return pltpu.make_async_copy(table_hbm.at[idx_v.at[c]], rows[b], gsems[b])

    def scatter(c, b):
        r = chunk_base + c
        tr = r // (32 * 8)
        tc = (r % (32 * 8)) // 8
        s = r % 8
        j = tr * 8 + s
        return pltpu.make_async_copy(
            tiles[b].at[:, :, pl.ds(0, CHUNK)], out_hbm.at[j, :, tc], ssems[b]
        )

    gather(0, 0).start()

    def outer(g0):
        for b in range(NBUF):
            c = g0 * NBUF + b
            nb = (b + 1) % NBUF

            @pl.when(c + 1 < n_chunks)
            def _():
                @pl.when(c + 1 >= NBUF)
                def _():
                    scatter(c + 1 - NBUF, nb).wait()
                gather(c + 1, nb).start()

            gather(c, b).wait()

            lanes = lax.iota(jnp.int32, NUM_LANES)
            dbv = [(jd * NUM_LANES + lanes) // 8 for jd in range(VECS)]
            dsv = [(jd * NUM_LANES + lanes) % 8 for jd in range(VECS)]

            @plsc.parallel_loop(0, CHUNK, step=UNROLL, unroll=4)
            def _(a):
                av = lanes * 0 + a
                for u in range(UNROLL):
                    for jd in range(VECS):
                        v = rows[b][a + u, pl.ds(jd * NUM_LANES, NUM_LANES)]
                        plsc.store_scatter(
                            tiles[b], [dbv[jd], dsv[jd], av + u], v * SCALE
                        )

            scatter(c, b).start()

    pl.loop(0, n_chunks // NBUF)(outer)

    for b in range(NBUF):
        scatter(n_chunks - NBUF + b, b).wait()


RB = 200
RNBUF = 2


def _repack_kernel(n_rows, table_hbm, out_hbm, inbufs, outbufs, isems, osems):
    wid = lax.axis_index("s") * NUM_CORES + lax.axis_index("c")
    n_blocks = n_rows // RB
    lanes = lax.iota(jnp.int32, NUM_LANES)

    def fetch(blk, b):
        return pltpu.make_async_copy(
            table_hbm.at[pl.ds(blk * RB, RB)], inbufs[b], isems[b]
        )

    def flush(blk, b):
        return pltpu.make_async_copy(
            outbufs[b], out_hbm.at[pl.ds(blk * RB, RB)], osems[b]
        )

    @pl.when(wid < n_blocks)
    def _():
        fetch(wid, 0).start()

    n_iter = (n_blocks + NW - 1) // NW

    def outer(i2):
        for b in range(RNBUF):
            i = i2 * RNBUF + b
            blk = i * NW + wid
            nb = (b + 1) % RNBUF

            @pl.when((i >= 1) & (blk - NW < n_blocks))
            def _():
                flush(blk - NW, nb).wait()

            @pl.when(blk + NW < n_blocks)
            def _():
                fetch(blk + NW, nb).start()

            @pl.when(blk < n_blocks)
            def _():
                fetch(blk, b).wait()

                @plsc.parallel_loop(0, RB, step=2, unroll=2)
                def _(a):
                    for u in range(2):
                        for jd in range(VECS):
                            sl = pl.ds(jd * NUM_LANES, NUM_LANES)
                            outbufs[b][a + u, sl] = inbufs[b][a + u, sl]

                flush(blk, b).start()

    n_outer = (n_iter + RNBUF - 1) // RNBUF
    pl.loop(0, n_outer)(outer)

    i_max = n_outer * RNBUF - 1

    @pl.when(i_max * NW + wid < n_blocks)
    def _():
        flush(i_max * NW + wid, i_max % RNBUF).wait()


def kernel(x, lut):
    n, m = x.shape
    b = n * m
    n_chunks = b // (NW * CHUNK)

    idx2 = (
        x.T.reshape(m // 8, 8, n // CHUNK, CHUNK)
        .transpose(0, 2, 1, 3)
        .reshape(b // CHUNK, CHUNK)
    )

    v = lut.shape[0]
    mesh = plsc.VectorSubcoreMesh(core_axis_name="c", subcore_axis_name="s")
    repack = pl.kernel(
        functools.partial(_repack_kernel, v),
        out_type=jax.ShapeDtypeStruct((v, CHUNK), jnp.float32),
        mesh=mesh,
        scratch_types=[
            [pltpu.VMEM((RB, D_MODEL), jnp.float32) for _ in range(RNBUF)],
            [pltpu.VMEM((RB, CHUNK), jnp.float32) for _ in range(RNBUF)],
            [pltpu.SemaphoreType.DMA for _ in range(RNBUF)],
            [pltpu.SemaphoreType.DMA for _ in range(RNBUF)],
        ],
        compiler_params=pltpu.CompilerParams(use_tc_tiling_on_sc=True),
    )
    lut_pad = repack(lut).reshape(-1, D_MODEL)
    run = pl.kernel(
        functools.partial(_emb_kernel, n_chunks),
        out_type=jax.ShapeDtypeStruct(
            (m, D_MODEL // 8, n // CHUNK, 8, CHUNK), jnp.float32
        ),
        mesh=mesh,
        scratch_types=[
            pltpu.VMEM((n_chunks, CHUNK), jnp.int32),
            [pltpu.VMEM((CHUNK, D_MODEL), jnp.float32) for _ in range(NBUF)],
            [pltpu.VMEM((D_MODEL // 8, 8, CHUNK + 1), jnp.float32) for _ in range(NBUF)],
            [pltpu.SemaphoreType.DMA for _ in range(NBUF)],
            [pltpu.SemaphoreType.DMA for _ in range(NBUF)],
        ],
        compiler_params=pltpu.CompilerParams(
            use_tc_tiling_on_sc=False, needs_layout_passes=False
        ),
    )
    out6 = run(idx2, lut_pad)
    return out6.transpose(2, 4, 0, 1, 3).reshape(n, m, D_MODEL)


# speedup vs baseline: 1.2336x; 1.2336x over previous
import functools
import math

import jax
import jax.numpy as jnp
from jax import lax
from jax.experimental import pallas as pl
from jax.experimental.pallas import tpu as pltpu
from jax.experimental.pallas import tpu_sc as plsc

D_MODEL = 64
SCALE = math.sqrt(D_MODEL)

NUM_CORES = 2
NUM_SUBCORES = 16
NUM_LANES = 16
NW = NUM_CORES * NUM_SUBCORES

CHUNK = 128
NBUF = 4
VECS = D_MODEL // NUM_LANES
UNROLL = 2


def _emb_kernel(n_chunks, idx_hbm, table_hbm, out_hbm, idx_v, rows, tiles, gsems, ssems):
    wid = lax.axis_index("s") * NUM_CORES + lax.axis_index("c")
    chunk_base = wid * n_chunks

    pltpu.sync_copy(idx_hbm.at[pl.ds(chunk_base, n_chunks)], idx_v)

    @plsc.parallel_loop(0, n_chunks, unroll=4)
    def _(g):
        for jd in range(CHUNK // NUM_LANES):
            sl = pl.ds(jd * NUM_LANES, NUM_LANES)
            idx_v[g, sl] = idx_v[g, sl] * 2

    def gather(c, b):
        return pltpu.make_async_copy(table_hbm.at[idx_v.at[c]], rows[b], gsems[b])

    def scatter(c, b):
        r = chunk_base + c
        tr = r // (32 * 8)
        tc = (r % (32 * 8)) // 8
        s = r % 8
        j = tr * 8 + s
        return pltpu.make_async_copy(
            tiles[b].at[:, :, pl.ds(0, CHUNK)], out_hbm.at[j, :, tc], ssems[b]
        )

    gather(0, 0).start()

    def outer(g0):
        for b in range(NBUF):
            c = g0 * NBUF + b
            nb = (b + 1) % NBUF

            @pl.when(c + 1 < n_chunks)
            def _():
                @pl.when(c + 1 >= NBUF)
                def _():
                    scatter(c + 1 - NBUF, nb).wait()
                gather(c + 1, nb).start()

            gather(c, b).wait()

            lanes = lax.iota(jnp.int32, NUM_LANES)
            dbv = [(jd * NUM_LANES + lanes) // 8 for jd in range(VECS)]
            dsv = [(jd * NUM_LANES + lanes) % 8 for jd in range(VECS)]

            @plsc.parallel_loop(0, CHUNK, step=UNROLL, unroll=4)
            def _(a):
                av = lanes * 0 + a
                for u in range(UNROLL):
                    for jd in range(VECS):
                        v = rows[b][a + u, pl.ds(jd * NUM_LANES, NUM_LANES)]
                        plsc.store_scatter(
                            tiles[b], [dbv[jd], dsv[jd], av + u], v * SCALE
                        )

            scatter(c, b).start()

    pl.loop(0, n_chunks // NBUF)(outer)

    for b in range(NBUF):
        scatter(n_chunks - NBUF + b, b).wait()


def kernel(x, lut):
    n, m = x.shape
    b = n * m
    n_chunks = b // (NW * CHUNK)

    idx2 = (
        x.T.reshape(m // 8, 8, n // CHUNK, CHUNK)
        .transpose(0, 2, 1, 3)
        .reshape(b // CHUNK, CHUNK)
    )

    lut_pad = jnp.pad(lut, ((0, 0), (0, CHUNK - D_MODEL))).reshape(-1, D_MODEL)
    mesh = plsc.VectorSubcoreMesh(core_axis_name="c", subcore_axis_name="s")
    run = pl.kernel(
        functools.partial(_emb_kernel, n_chunks),
        out_type=jax.ShapeDtypeStruct(
            (m, D_MODEL // 8, n // CHUNK, 8, CHUNK), jnp.float32
        ),
        mesh=mesh,
        scratch_types=[
            pltpu.VMEM((n_chunks, CHUNK), jnp.int32),
            [pltpu.VMEM((CHUNK, D_MODEL), jnp.float32) for _ in range(NBUF)],
            [pltpu.VMEM((D_MODEL // 8, 8, CHUNK + 1), jnp.float32) for _ in range(NBUF)],
            [pltpu.SemaphoreType.DMA for _ in range(NBUF)],
            [pltpu.SemaphoreType.DMA for _ in range(NBUF)],
        ],
        compiler_params=pltpu.CompilerParams(
            use_tc_tiling_on_sc=False, needs_layout_passes=False
        ),
    )
    out6 = run(idx2, lut_pad)
    return out6.transpose(2, 4, 0, 1, 3).reshape(n, m, D_MODEL)
